# Initial kernel scaffold; baseline (speedup 1.0000x reference)
import jax
import jax.numpy as jnp
from jax.experimental import pallas as pl


def kernel(boxes, scores):
    raise NotImplementedError("write your pallas kernel here")


# speedup vs baseline: 63.3951x; 63.3951x over previous
import jax
import jax.numpy as jnp
from jax.experimental import pallas as pl
from jax.experimental.pallas import tpu as pltpu

_N = 5000
_B = 128
_NB = 40
_NP = _NB * _B
_TH = 0.7


def _nms_body(x1c, y1c, x2c, y2c, x1r, y1r, x2r, y2r, out_ref, m_scr):
    out_ref[:, :] = jnp.zeros((_NB, _B), jnp.float32)
    lane = jax.lax.broadcasted_iota(jnp.int32, (1, _B), 1)
    jlt = (jax.lax.broadcasted_iota(jnp.int32, (_B, _B), 0)
           < jax.lax.broadcasted_iota(jnp.int32, (_B, _B), 1))

    def blk(b, carry):
        base = b * _B
        cx1 = x1c[pl.ds(base, _B), :]
        cy1 = y1c[pl.ds(base, _B), :]
        cx2 = x2c[pl.ds(base, _B), :]
        cy2 = y2c[pl.ds(base, _B), :]
        ac = (jnp.maximum(cx2 - cx1, 0.0) * jnp.maximum(cy2 - cy1, 0.0))

        rx1 = x1r[pl.ds(b, 1), :]
        ry1 = y1r[pl.ds(b, 1), :]
        rx2 = x2r[pl.ds(b, 1), :]
        ry2 = y2r[pl.ds(b, 1), :]
        ar = (jnp.maximum(rx2 - rx1, 0.0) * jnp.maximum(ry2 - ry1, 0.0))

        w = jnp.maximum(jnp.minimum(cx2, rx2) - jnp.maximum(cx1, rx1), 0.0)
        h = jnp.maximum(jnp.minimum(cy2, ry2) - jnp.maximum(cy1, ry1), 0.0)
        inter = w * h
        union = ac + ar - inter
        iou = inter / jnp.maximum(union, 1e-8)
        m_scr[:, :] = jnp.where((iou > _TH) & jlt, 1.0, 0.0)

        rem0 = out_ref[pl.ds(b, 1), :]

        def scan_body(i, rem):
            rowm = m_scr[pl.ds(i, 1), :]
            remi = jnp.sum(rem * (lane == i).astype(jnp.float32),
                           axis=1, keepdims=True)
            return jnp.maximum(rem, rowm * (1.0 - remi))

        rem = jax.lax.fori_loop(0, _B, scan_body, rem0)
        out_ref[pl.ds(b, 1), :] = rem
        keep = 1.0 - rem

        def cross(c, carry2):
            vx1 = x1r[pl.ds(c, 1), :]
            vy1 = y1r[pl.ds(c, 1), :]
            vx2 = x2r[pl.ds(c, 1), :]
            vy2 = y2r[pl.ds(c, 1), :]
            av = (jnp.maximum(vx2 - vx1, 0.0) * jnp.maximum(vy2 - vy1, 0.0))
            wv = jnp.maximum(jnp.minimum(cx2, vx2) - jnp.maximum(cx1, vx1), 0.0)
            hv = jnp.maximum(jnp.minimum(cy2, vy2) - jnp.maximum(cy1, vy1), 0.0)
            iv = wv * hv
            uv = ac + av - iv
            iouv = iv / jnp.maximum(uv, 1e-8)
            maskv = jnp.where(iouv > _TH, 1.0, 0.0)
            counts = jax.lax.dot_general(
                keep, maskv, (((1,), (0,)), ((), ())),
                preferred_element_type=jnp.float32)
            cur = out_ref[pl.ds(c, 1), :]
            out_ref[pl.ds(c, 1), :] = jnp.maximum(
                cur, jnp.where(counts >= 0.5, 1.0, 0.0))
            return carry2

        jax.lax.fori_loop(b + 1, _NB, cross, 0)
        return carry

    jax.lax.fori_loop(0, _NB, blk, 0)


def kernel(boxes, scores):
    order = jnp.argsort(-scores)
    b = jnp.take(boxes, order, axis=0)
    s = jnp.take(scores, order)
    bp = jnp.pad(b, ((0, _NP - _N), (0, 0)))
    x1c = bp[:, 0:1]
    y1c = bp[:, 1:2]
    x2c = bp[:, 2:3]
    y2c = bp[:, 3:4]
    x1r = bp[:, 0].reshape(_NB, _B)
    y1r = bp[:, 1].reshape(_NB, _B)
    x2r = bp[:, 2].reshape(_NB, _B)
    y2r = bp[:, 3].reshape(_NB, _B)
    removed = pl.pallas_call(
        _nms_body,
        out_shape=jax.ShapeDtypeStruct((_NB, _B), jnp.float32),
        scratch_shapes=[pltpu.VMEM((_B, _B), jnp.float32)],
    )(x1c, y1c, x2c, y2c, x1r, y1r, x2r, y2r)
    keep = 1.0 - removed.reshape(_NP)[:_N]
    proposals = jnp.concatenate([b, s[:, None]], axis=1) * keep[:, None]
    return proposals
